# baseline (speedup 1.0000x reference)
import jax
import jax.numpy as jnp
from jax.experimental import pallas as pl
from jax.experimental.pallas import tpu as pltpu

BN_EPS = 1e-3
F32 = jnp.float32


def _se_conv_kernel(x63_ref, w1t_ref, b1_ref, w2t_ref, x_ref,
                    y_ref, sum_ref, ssq_ref):
    b = pl.program_id(0)

    s_row = jax.nn.sigmoid(
        jnp.dot(x63_ref[b], w1t_ref[...], preferred_element_type=F32)
        + b1_ref[...])
    w2st = w2t_ref[...] * jnp.transpose(s_row)
    y = jnp.dot(x_ref[0], w2st, preferred_element_type=F32)
    y_ref[0] = y.astype(y_ref.dtype)

    @pl.when(b == 0)
    def _init():
        sum_ref[...] = jnp.zeros_like(sum_ref)
        ssq_ref[...] = jnp.zeros_like(ssq_ref)

    sum_ref[0] += jnp.sum(y, axis=0, keepdims=True)
    ssq_ref[0] += jnp.sum(y * y, axis=0, keepdims=True)


def _bn_apply_kernel(y_ref, scale_ref, shift_ref, out_ref):
    o = y_ref[0].astype(F32) * scale_ref[0] + shift_ref[0]
    out_ref[0] = o.astype(out_ref.dtype)


def kernel(x63, x60, w1, b1, w2, gamma, beta):
    N, C_mid, H, W = x60.shape
    C_se = x63.shape[1]
    C_out = w2.shape[0]
    hw = H * W

    xt = jnp.transpose(x60.astype(F32), (0, 2, 3, 1)).reshape(N, hw, C_mid)

    se_pad = 128 if C_se <= 128 else ((C_se + 127) // 128) * 128
    x63_p = jnp.pad(x63.astype(F32).reshape(N, 1, C_se),
                    ((0, 0), (0, 0), (0, se_pad - C_se)))
    w1t_p = jnp.pad(w1.astype(F32).T, ((0, se_pad - C_se), (0, 0)))
    b1_row = b1.astype(F32).reshape(1, C_mid)
    w2t = w2.astype(F32).T

    y, psum, pssq = pl.pallas_call(
        _se_conv_kernel,
        grid=(N,),
        in_specs=[
            pl.BlockSpec((N, 1, se_pad), lambda b: (0, 0, 0)),
            pl.BlockSpec((se_pad, C_mid), lambda b: (0, 0)),
            pl.BlockSpec((1, C_mid), lambda b: (0, 0)),
            pl.BlockSpec((C_mid, C_out), lambda b: (0, 0)),
            pl.BlockSpec((1, hw, C_mid), lambda b: (b, 0, 0)),
        ],
        out_specs=[
            pl.BlockSpec((1, hw, C_out), lambda b: (b, 0, 0)),
            pl.BlockSpec((1, 1, C_out), lambda b: (0, 0, 0)),
            pl.BlockSpec((1, 1, C_out), lambda b: (0, 0, 0)),
        ],
        out_shape=[
            jax.ShapeDtypeStruct((N, hw, C_out), jnp.bfloat16),
            jax.ShapeDtypeStruct((1, 1, C_out), F32),
            jax.ShapeDtypeStruct((1, 1, C_out), F32),
        ],
        compiler_params=pltpu.CompilerParams(
            dimension_semantics=("arbitrary",)),
    )(x63_p, w1t_p, b1_row, w2t, xt)

    n_elems = jnp.asarray(N * hw, F32)
    mean = psum / n_elems
    var = jnp.maximum(pssq / n_elems - mean * mean, 0.0)
    inv = jax.lax.rsqrt(var + BN_EPS)
    gamma3 = gamma.astype(F32).reshape(1, 1, C_out)
    scale = gamma3 * inv
    shift = beta.astype(F32).reshape(1, 1, C_out) - mean * scale

    out_nhwc = pl.pallas_call(
        _bn_apply_kernel,
        grid=(N,),
        in_specs=[
            pl.BlockSpec((1, hw, C_out), lambda b: (b, 0, 0)),
            pl.BlockSpec((1, 1, C_out), lambda b: (0, 0, 0)),
            pl.BlockSpec((1, 1, C_out), lambda b: (0, 0, 0)),
        ],
        out_specs=pl.BlockSpec((1, hw, C_out), lambda b: (b, 0, 0)),
        out_shape=jax.ShapeDtypeStruct((N, hw, C_out), jnp.bfloat16),
        compiler_params=pltpu.CompilerParams(
            dimension_semantics=("parallel",)),
    )(y, scale, shift)

    out = jnp.transpose(out_nhwc.reshape(N, H, W, C_out),
                        (0, 3, 1, 2)).astype(F32)
    return out


# speedup vs baseline: 1.1484x; 1.1484x over previous
import jax
import jax.numpy as jnp
from jax.experimental import pallas as pl
from jax.experimental.pallas import tpu as pltpu

BN_EPS = 1e-3
F32 = jnp.float32


def _se_conv_kernel(x63_ref, w1t_ref, b1_ref, w2t_ref, x_ref,
                    y_ref, sum_ref, ssq_ref):
    b = pl.program_id(0)

    s_row = jax.nn.sigmoid(
        jnp.dot(x63_ref[b], w1t_ref[...], preferred_element_type=F32)
        + b1_ref[...])
    w2st = w2t_ref[...] * jnp.transpose(s_row)
    y = jnp.dot(x_ref[0], w2st, preferred_element_type=F32)
    y_ref[0] = y.astype(y_ref.dtype)

    @pl.when(b == 0)
    def _init():
        sum_ref[...] = jnp.zeros_like(sum_ref)
        ssq_ref[...] = jnp.zeros_like(ssq_ref)

    sum_ref[0] += jnp.sum(y, axis=0, keepdims=True)
    ssq_ref[0] += jnp.sum(y * y, axis=0, keepdims=True)


def _bn_apply_kernel(y_ref, scale_ref, shift_ref, out_ref):
    o = y_ref[0].astype(F32) * scale_ref[0] + shift_ref[0]
    out_ref[0] = o.astype(out_ref.dtype)


def kernel(x63, x60, w1, b1, w2, gamma, beta):
    N, C_mid, H, W = x60.shape
    C_se = x63.shape[1]
    C_out = w2.shape[0]
    hw = H * W

    xt = jnp.transpose(x60.astype(F32), (0, 2, 3, 1)).reshape(N, hw, C_mid)

    se_pad = 128 if C_se <= 128 else ((C_se + 127) // 128) * 128
    x63_p = jnp.pad(x63.astype(F32).reshape(N, 1, C_se),
                    ((0, 0), (0, 0), (0, se_pad - C_se)))
    w1t_p = jnp.pad(w1.astype(F32).T, ((0, se_pad - C_se), (0, 0)))
    b1_row = b1.astype(F32).reshape(1, C_mid)
    w2t = w2.astype(F32).T

    y, psum, pssq = pl.pallas_call(
        _se_conv_kernel,
        grid=(N,),
        in_specs=[
            pl.BlockSpec((N, 1, se_pad), lambda b: (0, 0, 0)),
            pl.BlockSpec((se_pad, C_mid), lambda b: (0, 0)),
            pl.BlockSpec((1, C_mid), lambda b: (0, 0)),
            pl.BlockSpec((C_mid, C_out), lambda b: (0, 0)),
            pl.BlockSpec((1, hw, C_mid), lambda b: (b, 0, 0)),
        ],
        out_specs=[
            pl.BlockSpec((1, hw, C_out), lambda b: (b, 0, 0)),
            pl.BlockSpec((1, 1, C_out), lambda b: (0, 0, 0)),
            pl.BlockSpec((1, 1, C_out), lambda b: (0, 0, 0)),
        ],
        out_shape=[
            jax.ShapeDtypeStruct((N, hw, C_out), jnp.bfloat16),
            jax.ShapeDtypeStruct((1, 1, C_out), F32),
            jax.ShapeDtypeStruct((1, 1, C_out), F32),
        ],
        compiler_params=pltpu.CompilerParams(
            dimension_semantics=("arbitrary",)),
    )(x63_p, w1t_p, b1_row, w2t, xt)

    n_elems = jnp.asarray(N * hw, F32)
    mean = psum / n_elems
    var = jnp.maximum(pssq / n_elems - mean * mean, 0.0)
    inv = jax.lax.rsqrt(var + BN_EPS)
    gamma3 = gamma.astype(F32).reshape(1, 1, C_out)
    scale = gamma3 * inv
    shift = beta.astype(F32).reshape(1, 1, C_out) - mean * scale

    out_nhwc = pl.pallas_call(
        _bn_apply_kernel,
        grid=(N,),
        in_specs=[
            pl.BlockSpec((1, hw, C_out), lambda b: (b, 0, 0)),
            pl.BlockSpec((1, 1, C_out), lambda b: (0, 0, 0)),
            pl.BlockSpec((1, 1, C_out), lambda b: (0, 0, 0)),
        ],
        out_specs=pl.BlockSpec((1, hw, C_out), lambda b: (b, 0, 0)),
        out_shape=jax.ShapeDtypeStruct((N, hw, C_out), F32),
        compiler_params=pltpu.CompilerParams(
            dimension_semantics=("parallel",)),
    )(y, scale, shift)

    out = jnp.transpose(out_nhwc.reshape(N, H, W, C_out),
                        (0, 3, 1, 2)).astype(F32)
    return out


# speedup vs baseline: 1.2317x; 1.0725x over previous
import jax
import jax.numpy as jnp
from jax.experimental import pallas as pl
from jax.experimental.pallas import tpu as pltpu

BN_EPS = 1e-3
F32 = jnp.float32


def _se_conv_kernel(x63_ref, w1t_ref, b1_ref, w2t_ref, x_ref,
                    y_ref, sum_ref, ssq_ref):
    b = pl.program_id(0)

    s_row = jax.nn.sigmoid(
        jnp.dot(x63_ref[b], w1t_ref[...], preferred_element_type=F32)
        + b1_ref[...])
    w2st = w2t_ref[...] * jnp.transpose(s_row)
    y = jnp.dot(x_ref[0], w2st, preferred_element_type=F32)
    y_ref[0] = y

    @pl.when(b == 0)
    def _init():
        sum_ref[...] = jnp.zeros_like(sum_ref)
        ssq_ref[...] = jnp.zeros_like(ssq_ref)

    sum_ref[0] += jnp.sum(y, axis=0, keepdims=True)
    ssq_ref[0] += jnp.sum(y * y, axis=0, keepdims=True)


def _bn_apply_kernel(y_ref, scale_ref, shift_ref, out_ref):
    c_out, h, w = out_ref.shape[1:]
    o = y_ref[0] * scale_ref[0] + shift_ref[0]
    out_ref[0] = o.reshape(c_out, h, w)


def kernel(x63, x60, w1, b1, w2, gamma, beta):
    N, C_mid, H, W = x60.shape
    C_se = x63.shape[1]
    C_out = w2.shape[0]
    hw = H * W

    xt = jnp.transpose(x60.astype(F32), (0, 2, 3, 1)).reshape(N, hw, C_mid)

    se_pad = 128 if C_se <= 128 else ((C_se + 127) // 128) * 128
    x63_p = jnp.pad(x63.astype(F32).reshape(N, 1, C_se),
                    ((0, 0), (0, 0), (0, se_pad - C_se)))
    w1t_p = jnp.pad(w1.astype(F32).T, ((0, se_pad - C_se), (0, 0)))
    b1_row = b1.astype(F32).reshape(1, C_mid)
    w2t = w2.astype(F32).T

    y, psum, pssq = pl.pallas_call(
        _se_conv_kernel,
        grid=(N,),
        in_specs=[
            pl.BlockSpec((N, 1, se_pad), lambda b: (0, 0, 0)),
            pl.BlockSpec((se_pad, C_mid), lambda b: (0, 0)),
            pl.BlockSpec((1, C_mid), lambda b: (0, 0)),
            pl.BlockSpec((C_mid, C_out), lambda b: (0, 0)),
            pl.BlockSpec((1, hw, C_mid), lambda b: (b, 0, 0)),
        ],
        out_specs=[
            pl.BlockSpec((1, hw, C_out), lambda b: (b, 0, 0)),
            pl.BlockSpec((1, 1, C_out), lambda b: (0, 0, 0)),
            pl.BlockSpec((1, 1, C_out), lambda b: (0, 0, 0)),
        ],
        out_shape=[
            jax.ShapeDtypeStruct((N, hw, C_out), F32),
            jax.ShapeDtypeStruct((1, 1, C_out), F32),
            jax.ShapeDtypeStruct((1, 1, C_out), F32),
        ],
        compiler_params=pltpu.CompilerParams(
            dimension_semantics=("arbitrary",)),
    )(x63_p, w1t_p, b1_row, w2t, xt)

    y_chw = jnp.transpose(y, (0, 2, 1))

    n_elems = jnp.asarray(N * hw, F32)
    mean = psum.reshape(C_out, 1) / n_elems
    var = jnp.maximum(pssq.reshape(C_out, 1) / n_elems - mean * mean, 0.0)
    inv = jax.lax.rsqrt(var + BN_EPS)
    gammac = gamma.astype(F32).reshape(C_out, 1)
    scale = (gammac * inv).reshape(1, C_out, 1)
    shift = (beta.astype(F32).reshape(C_out, 1)
             - mean * gammac * inv).reshape(1, C_out, 1)

    out = pl.pallas_call(
        _bn_apply_kernel,
        grid=(N,),
        in_specs=[
            pl.BlockSpec((1, C_out, hw), lambda b: (b, 0, 0)),
            pl.BlockSpec((1, C_out, 1), lambda b: (0, 0, 0)),
            pl.BlockSpec((1, C_out, 1), lambda b: (0, 0, 0)),
        ],
        out_specs=pl.BlockSpec((1, C_out, H, W), lambda b: (b, 0, 0, 0)),
        out_shape=jax.ShapeDtypeStruct((N, C_out, H, W), F32),
        compiler_params=pltpu.CompilerParams(
            dimension_semantics=("parallel",)),
    )(y_chw, scale, shift)

    return out


# speedup vs baseline: 1.4141x; 1.1481x over previous
import jax
import jax.numpy as jnp
from jax.experimental import pallas as pl
from jax.experimental.pallas import tpu as pltpu

BN_EPS = 1e-3
F32 = jnp.float32


def _se_conv_kernel(x63_ref, w1t_ref, b1_ref, w2t_ref, x_ref,
                    y_ref, sum_ref, ssq_ref):
    b = pl.program_id(0)

    s_row = jax.nn.sigmoid(
        jnp.dot(x63_ref[b], w1t_ref[...], preferred_element_type=F32)
        + b1_ref[...])
    w2st = w2t_ref[...] * jnp.transpose(s_row)
    y = jnp.dot(x_ref[0], w2st, preferred_element_type=F32)
    y_ref[0] = y.astype(y_ref.dtype)

    @pl.when(b == 0)
    def _init():
        sum_ref[...] = jnp.zeros_like(sum_ref)
        ssq_ref[...] = jnp.zeros_like(ssq_ref)

    sum_ref[0] += jnp.sum(y, axis=0, keepdims=True)
    ssq_ref[0] += jnp.sum(y * y, axis=0, keepdims=True)


def _bn_apply_kernel(y_ref, scale_ref, shift_ref, out_ref):
    c_out, h, w = out_ref.shape[1:]
    o = y_ref[0].astype(F32) * scale_ref[0] + shift_ref[0]
    out_ref[0] = o.reshape(c_out, h, w)


def kernel(x63, x60, w1, b1, w2, gamma, beta):
    N, C_mid, H, W = x60.shape
    C_se = x63.shape[1]
    C_out = w2.shape[0]
    hw = H * W

    xt = jnp.transpose(x60.astype(F32), (0, 2, 3, 1)).reshape(N, hw, C_mid)

    se_pad = 128 if C_se <= 128 else ((C_se + 127) // 128) * 128
    x63_p = jnp.pad(x63.astype(F32).reshape(N, 1, C_se),
                    ((0, 0), (0, 0), (0, se_pad - C_se)))
    w1t_p = jnp.pad(w1.astype(F32).T, ((0, se_pad - C_se), (0, 0)))
    b1_row = b1.astype(F32).reshape(1, C_mid)
    w2t = w2.astype(F32).T

    y, psum, pssq = pl.pallas_call(
        _se_conv_kernel,
        grid=(N,),
        in_specs=[
            pl.BlockSpec((N, 1, se_pad), lambda b: (0, 0, 0)),
            pl.BlockSpec((se_pad, C_mid), lambda b: (0, 0)),
            pl.BlockSpec((1, C_mid), lambda b: (0, 0)),
            pl.BlockSpec((C_mid, C_out), lambda b: (0, 0)),
            pl.BlockSpec((1, hw, C_mid), lambda b: (b, 0, 0)),
        ],
        out_specs=[
            pl.BlockSpec((1, hw, C_out), lambda b: (b, 0, 0)),
            pl.BlockSpec((1, 1, C_out), lambda b: (0, 0, 0)),
            pl.BlockSpec((1, 1, C_out), lambda b: (0, 0, 0)),
        ],
        out_shape=[
            jax.ShapeDtypeStruct((N, hw, C_out), jnp.bfloat16),
            jax.ShapeDtypeStruct((1, 1, C_out), F32),
            jax.ShapeDtypeStruct((1, 1, C_out), F32),
        ],
        compiler_params=pltpu.CompilerParams(
            dimension_semantics=("arbitrary",)),
    )(x63_p, w1t_p, b1_row, w2t, xt)

    y_chw = jnp.transpose(y, (0, 2, 1))

    n_elems = jnp.asarray(N * hw, F32)
    mean = psum.reshape(C_out, 1) / n_elems
    var = jnp.maximum(pssq.reshape(C_out, 1) / n_elems - mean * mean, 0.0)
    inv = jax.lax.rsqrt(var + BN_EPS)
    gammac = gamma.astype(F32).reshape(C_out, 1)
    scale = (gammac * inv).reshape(1, C_out, 1)
    shift = (beta.astype(F32).reshape(C_out, 1)
             - mean * gammac * inv).reshape(1, C_out, 1)

    out = pl.pallas_call(
        _bn_apply_kernel,
        grid=(N,),
        in_specs=[
            pl.BlockSpec((1, C_out, hw), lambda b: (b, 0, 0)),
            pl.BlockSpec((1, C_out, 1), lambda b: (0, 0, 0)),
            pl.BlockSpec((1, C_out, 1), lambda b: (0, 0, 0)),
        ],
        out_specs=pl.BlockSpec((1, C_out, H, W), lambda b: (b, 0, 0, 0)),
        out_shape=jax.ShapeDtypeStruct((N, C_out, H, W), F32),
        compiler_params=pltpu.CompilerParams(
            dimension_semantics=("parallel",)),
    )(y_chw, scale, shift)

    return out


# speedup vs baseline: 1.6114x; 1.1395x over previous
import jax
import jax.numpy as jnp
from jax.experimental import pallas as pl
from jax.experimental.pallas import tpu as pltpu

BN_EPS = 1e-3
F32 = jnp.float32


def _se_conv_kernel(x63_ref, w1t_ref, b1_ref, w2t_ref, x_ref,
                    y_ref, sum_ref, ssq_ref):
    g = pl.program_id(0)
    n_img = x_ref.shape[0]

    @pl.when(g == 0)
    def _init():
        sum_ref[...] = jnp.zeros_like(sum_ref)
        ssq_ref[...] = jnp.zeros_like(ssq_ref)

    for j in range(n_img):
        b = g * n_img + j
        s_row = jax.nn.sigmoid(
            jnp.dot(x63_ref[b], w1t_ref[...], preferred_element_type=F32)
            + b1_ref[...])
        w2st = w2t_ref[...] * jnp.transpose(s_row)
        y = jnp.dot(x_ref[j], w2st,
                    preferred_element_type=F32)
        y_ref[j] = y.astype(y_ref.dtype)
        sum_ref[0] += jnp.sum(y, axis=0, keepdims=True)
        ssq_ref[0] += jnp.sum(y * y, axis=0, keepdims=True)


def _bn_apply_kernel(y_ref, scale_ref, shift_ref, out_ref):
    c_out, h, w = out_ref.shape[1:]
    o = y_ref[0].astype(F32) * scale_ref[0] + shift_ref[0]
    out_ref[0] = o.reshape(c_out, h, w)


def kernel(x63, x60, w1, b1, w2, gamma, beta):
    N, C_mid, H, W = x60.shape
    C_se = x63.shape[1]
    C_out = w2.shape[0]
    hw = H * W

    xt = jnp.transpose(x60.astype(F32), (0, 2, 3, 1)).reshape(N, hw, C_mid)

    se_pad = 128 if C_se <= 128 else ((C_se + 127) // 128) * 128
    x63_p = jnp.pad(x63.astype(F32).reshape(N, 1, C_se),
                    ((0, 0), (0, 0), (0, se_pad - C_se)))
    w1t_p = jnp.pad(w1.astype(F32).T, ((0, se_pad - C_se), (0, 0)))
    b1_row = b1.astype(F32).reshape(1, C_mid)
    w2t = w2.astype(F32).T

    n_img = 2 if N % 2 == 0 else 1
    y, psum, pssq = pl.pallas_call(
        _se_conv_kernel,
        grid=(N // n_img,),
        in_specs=[
            pl.BlockSpec((N, 1, se_pad), lambda b: (0, 0, 0)),
            pl.BlockSpec((se_pad, C_mid), lambda b: (0, 0)),
            pl.BlockSpec((1, C_mid), lambda b: (0, 0)),
            pl.BlockSpec((C_mid, C_out), lambda b: (0, 0)),
            pl.BlockSpec((n_img, hw, C_mid), lambda b: (b, 0, 0)),
        ],
        out_specs=[
            pl.BlockSpec((n_img, hw, C_out), lambda b: (b, 0, 0)),
            pl.BlockSpec((1, 1, C_out), lambda b: (0, 0, 0)),
            pl.BlockSpec((1, 1, C_out), lambda b: (0, 0, 0)),
        ],
        out_shape=[
            jax.ShapeDtypeStruct((N, hw, C_out), jnp.bfloat16),
            jax.ShapeDtypeStruct((1, 1, C_out), F32),
            jax.ShapeDtypeStruct((1, 1, C_out), F32),
        ],
        compiler_params=pltpu.CompilerParams(
            dimension_semantics=("arbitrary",)),
    )(x63_p, w1t_p, b1_row, w2t, xt)

    y_chw = jnp.transpose(y, (0, 2, 1))

    n_elems = jnp.asarray(N * hw, F32)
    mean = psum.reshape(C_out, 1) / n_elems
    var = jnp.maximum(pssq.reshape(C_out, 1) / n_elems - mean * mean, 0.0)
    inv = jax.lax.rsqrt(var + BN_EPS)
    gammac = gamma.astype(F32).reshape(C_out, 1)
    scale = (gammac * inv).reshape(1, C_out, 1)
    shift = (beta.astype(F32).reshape(C_out, 1)
             - mean * gammac * inv).reshape(1, C_out, 1)

    out = pl.pallas_call(
        _bn_apply_kernel,
        grid=(N,),
        in_specs=[
            pl.BlockSpec((1, C_out, hw), lambda b: (b, 0, 0)),
            pl.BlockSpec((1, C_out, 1), lambda b: (0, 0, 0)),
            pl.BlockSpec((1, C_out, 1), lambda b: (0, 0, 0)),
        ],
        out_specs=pl.BlockSpec((1, C_out, H, W), lambda b: (b, 0, 0, 0)),
        out_shape=jax.ShapeDtypeStruct((N, C_out, H, W), F32),
        compiler_params=pltpu.CompilerParams(
            dimension_semantics=("parallel",)),
    )(y_chw, scale, shift)

    return out


# speedup vs baseline: 1.7352x; 1.0768x over previous
import jax
import jax.numpy as jnp
from jax.experimental import pallas as pl
from jax.experimental.pallas import tpu as pltpu

BN_EPS = 1e-3
F32 = jnp.float32


def _se_conv_kernel(x63_ref, w1t_ref, b1_ref, w2t_ref, x_ref,
                    y_ref, sum_ref, ssq_ref):
    g = pl.program_id(0)
    n_img = x_ref.shape[0]

    @pl.when(g == 0)
    def _init():
        sum_ref[...] = jnp.zeros_like(sum_ref)
        ssq_ref[...] = jnp.zeros_like(ssq_ref)

    for j in range(n_img):
        b = g * n_img + j
        s_row = jax.nn.sigmoid(
            jnp.dot(x63_ref[b], w1t_ref[...], preferred_element_type=F32)
            + b1_ref[...])
        w2st = w2t_ref[...] * jnp.transpose(s_row)
        y = jnp.dot(x_ref[j], w2st,
                    preferred_element_type=F32)
        y_ref[j] = y.astype(y_ref.dtype)
        sum_ref[0] += jnp.sum(y, axis=0, keepdims=True)
        ssq_ref[0] += jnp.sum(y * y, axis=0, keepdims=True)


def _bn_apply_kernel(y_ref, scale_ref, shift_ref, out_ref):
    c_out, h, w = out_ref.shape[1:]
    for j in range(out_ref.shape[0]):
        o = y_ref[j].astype(F32) * scale_ref[0] + shift_ref[0]
        out_ref[j] = o.reshape(c_out, h, w)


def kernel(x63, x60, w1, b1, w2, gamma, beta):
    N, C_mid, H, W = x60.shape
    C_se = x63.shape[1]
    C_out = w2.shape[0]
    hw = H * W

    xt = jnp.transpose(x60.astype(F32), (0, 2, 3, 1)).reshape(N, hw, C_mid)

    se_pad = 128 if C_se <= 128 else ((C_se + 127) // 128) * 128
    x63_p = jnp.pad(x63.astype(F32).reshape(N, 1, C_se),
                    ((0, 0), (0, 0), (0, se_pad - C_se)))
    w1t_p = jnp.pad(w1.astype(F32).T, ((0, se_pad - C_se), (0, 0)))
    b1_row = b1.astype(F32).reshape(1, C_mid)
    w2t = w2.astype(F32).T

    n_img = 2 if N % 2 == 0 else 1
    y, psum, pssq = pl.pallas_call(
        _se_conv_kernel,
        grid=(N // n_img,),
        in_specs=[
            pl.BlockSpec((N, 1, se_pad), lambda b: (0, 0, 0)),
            pl.BlockSpec((se_pad, C_mid), lambda b: (0, 0)),
            pl.BlockSpec((1, C_mid), lambda b: (0, 0)),
            pl.BlockSpec((C_mid, C_out), lambda b: (0, 0)),
            pl.BlockSpec((n_img, hw, C_mid), lambda b: (b, 0, 0)),
        ],
        out_specs=[
            pl.BlockSpec((n_img, hw, C_out), lambda b: (b, 0, 0)),
            pl.BlockSpec((1, 1, C_out), lambda b: (0, 0, 0)),
            pl.BlockSpec((1, 1, C_out), lambda b: (0, 0, 0)),
        ],
        out_shape=[
            jax.ShapeDtypeStruct((N, hw, C_out), jnp.bfloat16),
            jax.ShapeDtypeStruct((1, 1, C_out), F32),
            jax.ShapeDtypeStruct((1, 1, C_out), F32),
        ],
        compiler_params=pltpu.CompilerParams(
            dimension_semantics=("arbitrary",)),
    )(x63_p, w1t_p, b1_row, w2t, xt)

    y_chw = jnp.transpose(y, (0, 2, 1))

    n_elems = jnp.asarray(N * hw, F32)
    mean = psum.reshape(C_out, 1) / n_elems
    var = jnp.maximum(pssq.reshape(C_out, 1) / n_elems - mean * mean, 0.0)
    inv = jax.lax.rsqrt(var + BN_EPS)
    gammac = gamma.astype(F32).reshape(C_out, 1)
    scale = (gammac * inv).reshape(1, C_out, 1)
    shift = (beta.astype(F32).reshape(C_out, 1)
             - mean * gammac * inv).reshape(1, C_out, 1)

    out = pl.pallas_call(
        _bn_apply_kernel,
        grid=(N // n_img,),
        in_specs=[
            pl.BlockSpec((n_img, C_out, hw), lambda b: (b, 0, 0)),
            pl.BlockSpec((1, C_out, 1), lambda b: (0, 0, 0)),
            pl.BlockSpec((1, C_out, 1), lambda b: (0, 0, 0)),
        ],
        out_specs=pl.BlockSpec((n_img, C_out, H, W),
                               lambda b: (b, 0, 0, 0)),
        out_shape=jax.ShapeDtypeStruct((N, C_out, H, W), F32),
        compiler_params=pltpu.CompilerParams(
            dimension_semantics=("parallel",)),
    )(y_chw, scale, shift)

    return out


# speedup vs baseline: 1.7739x; 1.0223x over previous
import jax
import jax.numpy as jnp
from jax.experimental import pallas as pl
from jax.experimental.pallas import tpu as pltpu

BN_EPS = 1e-3
F32 = jnp.float32


def _se_conv_kernel(x63_ref, w1t_ref, b1_ref, w2t_ref, x_ref,
                    y_ref, sum_ref, ssq_ref):
    g = pl.program_id(0)
    n_img = x_ref.shape[0]

    @pl.when(g == 0)
    def _init():
        sum_ref[...] = jnp.zeros_like(sum_ref)
        ssq_ref[...] = jnp.zeros_like(ssq_ref)

    for j in range(n_img):
        b = g * n_img + j
        s_row = jax.nn.sigmoid(
            jnp.dot(x63_ref[b], w1t_ref[...], preferred_element_type=F32)
            + b1_ref[...])
        w2st = w2t_ref[...] * jnp.transpose(s_row)
        y = jnp.dot(x_ref[j], w2st,
                    preferred_element_type=F32)
        y_ref[j] = y.astype(y_ref.dtype)
        sum_ref[0] += jnp.sum(y, axis=0, keepdims=True)
        ssq_ref[0] += jnp.sum(y * y, axis=0, keepdims=True)


def _bn_apply_kernel(y_ref, scale_ref, shift_ref, out_ref):
    c_out, h, w = out_ref.shape[1:]
    for j in range(out_ref.shape[0]):
        o = y_ref[j].astype(F32) * scale_ref[0] + shift_ref[0]
        out_ref[j] = o.reshape(c_out, h, w)


def kernel(x63, x60, w1, b1, w2, gamma, beta):
    N, C_mid, H, W = x60.shape
    C_se = x63.shape[1]
    C_out = w2.shape[0]
    hw = H * W

    xt = jnp.transpose(x60.astype(F32), (0, 2, 3, 1)).reshape(N, hw, C_mid)

    se_pad = 128 if C_se <= 128 else ((C_se + 127) // 128) * 128
    x63_p = jnp.pad(x63.astype(F32).reshape(N, 1, C_se),
                    ((0, 0), (0, 0), (0, se_pad - C_se)))
    w1t_p = jnp.pad(w1.astype(F32).T, ((0, se_pad - C_se), (0, 0)))
    b1_row = b1.astype(F32).reshape(1, C_mid)
    w2t = w2.astype(F32).T

    n_img = 4 if N % 4 == 0 else (2 if N % 2 == 0 else 1)
    y, psum, pssq = pl.pallas_call(
        _se_conv_kernel,
        grid=(N // n_img,),
        in_specs=[
            pl.BlockSpec((N, 1, se_pad), lambda b: (0, 0, 0)),
            pl.BlockSpec((se_pad, C_mid), lambda b: (0, 0)),
            pl.BlockSpec((1, C_mid), lambda b: (0, 0)),
            pl.BlockSpec((C_mid, C_out), lambda b: (0, 0)),
            pl.BlockSpec((n_img, hw, C_mid), lambda b: (b, 0, 0)),
        ],
        out_specs=[
            pl.BlockSpec((n_img, hw, C_out), lambda b: (b, 0, 0)),
            pl.BlockSpec((1, 1, C_out), lambda b: (0, 0, 0)),
            pl.BlockSpec((1, 1, C_out), lambda b: (0, 0, 0)),
        ],
        out_shape=[
            jax.ShapeDtypeStruct((N, hw, C_out), jnp.bfloat16),
            jax.ShapeDtypeStruct((1, 1, C_out), F32),
            jax.ShapeDtypeStruct((1, 1, C_out), F32),
        ],
        compiler_params=pltpu.CompilerParams(
            dimension_semantics=("arbitrary",)),
    )(x63_p, w1t_p, b1_row, w2t, xt)

    y_chw = jnp.transpose(y, (0, 2, 1))

    n_elems = jnp.asarray(N * hw, F32)
    mean = psum.reshape(C_out, 1) / n_elems
    var = jnp.maximum(pssq.reshape(C_out, 1) / n_elems - mean * mean, 0.0)
    inv = jax.lax.rsqrt(var + BN_EPS)
    gammac = gamma.astype(F32).reshape(C_out, 1)
    scale = (gammac * inv).reshape(1, C_out, 1)
    shift = (beta.astype(F32).reshape(C_out, 1)
             - mean * gammac * inv).reshape(1, C_out, 1)

    out = pl.pallas_call(
        _bn_apply_kernel,
        grid=(N // n_img,),
        in_specs=[
            pl.BlockSpec((n_img, C_out, hw), lambda b: (b, 0, 0)),
            pl.BlockSpec((1, C_out, 1), lambda b: (0, 0, 0)),
            pl.BlockSpec((1, C_out, 1), lambda b: (0, 0, 0)),
        ],
        out_specs=pl.BlockSpec((n_img, C_out, H, W),
                               lambda b: (b, 0, 0, 0)),
        out_shape=jax.ShapeDtypeStruct((N, C_out, H, W), F32),
        compiler_params=pltpu.CompilerParams(
            dimension_semantics=("parallel",)),
    )(y_chw, scale, shift)

    return out
